# baseline (speedup 1.0000x reference)
import functools

import jax
import jax.numpy as jnp
from jax import lax
from jax.experimental import pallas as pl
from jax.experimental.pallas import tpu as pltpu
from jax.experimental.pallas import tpu_sc as plsc

NC = 2
NS = 16
NW = NC * NS
CHUNK = 128
R = 512


def _sc_mesh():
    return plsc.VectorSubcoreMesh(core_axis_name="c", subcore_axis_name="s")


BI = 5


def _make_deg_kernel(e_iters, n_pad):
    iters = e_iters // NW
    nb = iters // BI
    rows_pt = n_pad // NS

    @functools.partial(
        pl.kernel,
        mesh=_sc_mesh(),
        out_type=jax.ShapeDtypeStruct((NC, NS, 1, rows_pt), jnp.float32),
        scratch_types=[
            pltpu.VMEM((BI, CHUNK), jnp.int32),
            pltpu.VMEM((CHUNK,), jnp.float32),
            pltpu.VMEM((rows_pt,), jnp.float32),
            pltpu.VMEM_SHARED((n_pad,), jnp.float32),
            pltpu.SemaphoreType.DMA,
            pltpu.SemaphoreType.DMA,
        ],
    )
    def deg_kernel(dst_hbm, out_hbm, dst_v, ones_v, stage_v, acc_sh, sem_a, sem_b):
        c = lax.axis_index("c")
        s = lax.axis_index("s")
        wid = c * NS + s

        def fill_ones(k, _):
            ones_v[pl.ds(k * 16, 16)] = jnp.ones((16,), jnp.float32)
            return 0

        lax.fori_loop(0, CHUNK // 16, fill_ones, 0)

        def zero_stage(k, _):
            stage_v[pl.ds(k * 16, 16)] = jnp.zeros((16,), jnp.float32)
            return 0

        lax.fori_loop(0, rows_pt // 16, zero_stage, 0)
        pltpu.sync_copy(stage_v, acc_sh.at[pl.ds(s * rows_pt, rows_pt)])
        plsc.subcore_barrier()

        def block(b, _):
            pltpu.sync_copy(dst_hbm.at[wid, b], dst_v)
            pltpu.async_copy(ones_v, acc_sh.at[dst_v.at[0]], sem_a, add=True)

            def body(t, __):
                j = 2 * t
                d1 = pltpu.async_copy(ones_v, acc_sh.at[dst_v.at[j + 1]],
                                      sem_b, add=True)
                pltpu.make_async_copy(ones_v, acc_sh.at[dst_v.at[j]], sem_a).wait()
                pltpu.async_copy(ones_v, acc_sh.at[dst_v.at[j + 2]], sem_a, add=True)
                d1.wait()
                return 0

            lax.fori_loop(0, (BI - 1) // 2, body, 0)
            pltpu.make_async_copy(ones_v, acc_sh.at[dst_v.at[BI - 1]], sem_a).wait()
            return 0

        lax.fori_loop(0, nb, block, 0)
        plsc.subcore_barrier()

        pltpu.sync_copy(acc_sh.at[pl.ds(s * rows_pt, rows_pt)], stage_v)
        pltpu.sync_copy(stage_v, out_hbm.at[c, s, 0])

    return deg_kernel


def _make_agg_kernel(e_iters, n_pad, d):
    iters = e_iters // NW
    rows_pt = n_pad // NS
    out_chunks = rows_pt // CHUNK

    nb = iters // BI

    @functools.partial(
        pl.kernel,
        mesh=_sc_mesh(),
        out_type=jax.ShapeDtypeStruct((NC, n_pad, d), jnp.float32),
        scratch_types=[
            pltpu.VMEM((BI, CHUNK), jnp.int32),
            pltpu.VMEM((BI, CHUNK), jnp.int32),
            pltpu.VMEM((CHUNK, d), jnp.float32),
            pltpu.VMEM((CHUNK, d), jnp.float32),
            pltpu.VMEM_SHARED((n_pad, d), jnp.float32),
            pltpu.SemaphoreType.DMA,
            pltpu.SemaphoreType.DMA,
        ],
    )
    def agg_kernel(g_hbm, src_hbm, dst_hbm, out_hbm,
                   src_v, dst_v, buf_a, buf_b, acc_sh, sem_a, sem_b):
        c = lax.axis_index("c")
        s = lax.axis_index("s")
        wid = c * NS + s
        r0 = s * rows_pt

        def zrow(r, _):
            def zcol(k, __):
                buf_a[r, pl.ds(k * 16, 16)] = jnp.zeros((16,), jnp.float32)
                return 0
            lax.fori_loop(0, d // 16, zcol, 0)
            return 0

        lax.fori_loop(0, CHUNK, zrow, 0)
        for k in range(out_chunks):
            pltpu.sync_copy(buf_a, acc_sh.at[pl.ds(r0 + k * CHUNK, CHUNK)])
        plsc.subcore_barrier()

        def block(b, _):
            pltpu.sync_copy(src_hbm.at[wid, b], src_v)
            pltpu.sync_copy(dst_hbm.at[wid, b], dst_v)
            pltpu.async_copy(g_hbm.at[src_v.at[0]], buf_a, sem_a)

            def body(t, __):
                j = 2 * t
                d1 = pltpu.async_copy(g_hbm.at[src_v.at[j + 1]], buf_b, sem_b)
                pltpu.make_async_copy(g_hbm.at[src_v.at[j]], buf_a, sem_a).wait()
                pltpu.sync_copy(buf_a, acc_sh.at[dst_v.at[j]], add=True)
                pltpu.async_copy(g_hbm.at[src_v.at[j + 2]], buf_a, sem_a)
                d1.wait()
                pltpu.sync_copy(buf_b, acc_sh.at[dst_v.at[j + 1]], add=True)
                return 0

            lax.fori_loop(0, (BI - 1) // 2, body, 0)
            pltpu.make_async_copy(g_hbm.at[src_v.at[BI - 1]], buf_a, sem_a).wait()
            pltpu.sync_copy(buf_a, acc_sh.at[dst_v.at[BI - 1]], add=True)
            return 0

        lax.fori_loop(0, nb, block, 0)
        plsc.subcore_barrier()

        for k in range(out_chunks):
            pltpu.sync_copy(acc_sh.at[pl.ds(r0 + k * CHUNK, CHUNK)], buf_a)
            pltpu.sync_copy(buf_a, out_hbm.at[c, pl.ds(r0 + k * CHUNK, CHUNK)])

    return agg_kernel


def _dis_from_parts(p_ref):
    deg = p_ref[0, :, :] + p_ref[1, :, :] + 1.0
    return lax.rsqrt(deg)


def _dense1_body(x_ref, w_ref, p_ref, o_ref):
    dis = _dis_from_parts(p_ref)
    h = jnp.dot(x_ref[...], w_ref[...], preferred_element_type=jnp.float32)
    o_ref[...] = h * dis


def _dense2_body(pp_ref, g_ref, p_ref, w_ref, b_ref, o_ref):
    dis = _dis_from_parts(p_ref)
    z = (pp_ref[0, :, :] + pp_ref[1, :, :] + g_ref[...]) * dis + b_ref[...]
    a = jnp.maximum(z, 0.0)
    o_ref[...] = jnp.dot(a, w_ref[...], preferred_element_type=jnp.float32) * dis


def _dense3_body(pp_ref, g_ref, p_ref, b_ref, wo_ref, bo_ref, o_ref):
    dis = _dis_from_parts(p_ref)
    z = (pp_ref[0, :, :] + pp_ref[1, :, :] + g_ref[...]) * dis + b_ref[...]
    a = jnp.maximum(z, 0.0)
    t = jnp.sum(a * wo_ref[...], axis=1, keepdims=True) + bo_ref[...]
    o_ref[...] = jax.nn.sigmoid(t)


def kernel(x, edge_index, W1, b1, W2, b2, Wo, bo):
    n, d = x.shape
    e = edge_index.shape[1]
    n_pad = -(-n // R) * R
    assert n_pad % (NS * CHUNK) == 0 and BI % 2 == 1
    unit = NW * CHUNK * BI
    e_pad = -(-e // unit) * unit
    iters = e_pad // (NW * CHUNK)
    nb = n_pad // R

    src1, dst1 = edge_index[0], edge_index[1]
    if e_pad > e:
        assert n_pad > n
        pad_idx = (n + jnp.arange(e_pad - e, dtype=jnp.int32) % (n_pad - n))
        src1 = jnp.concatenate([src1, pad_idx])
        dst1 = jnp.concatenate([dst1, pad_idx])
    src2 = src1.reshape(NW, iters // BI, BI, CHUNK)
    dst2 = dst1.reshape(NW, iters // BI, BI, CHUNK)
    x_pad = jnp.pad(x, ((0, n_pad - n), (0, 0)))

    parts = _make_deg_kernel(e_pad // CHUNK, n_pad)(dst2)
    parts3 = parts.reshape(NC, n_pad, 1)
    agg = _make_agg_kernel(e_pad // CHUNK, n_pad, d)

    full = lambda *idx: (lambda i: idx)
    g1 = pl.pallas_call(
        _dense1_body,
        grid=(nb,),
        in_specs=[
            pl.BlockSpec((R, d), lambda i: (i, 0)),
            pl.BlockSpec((d, d), full(0, 0)),
            pl.BlockSpec((NC, R, 1), lambda i: (0, i, 0)),
        ],
        out_specs=pl.BlockSpec((R, d), lambda i: (i, 0)),
        out_shape=jax.ShapeDtypeStruct((n_pad, d), jnp.float32),
    )(x_pad, W1, parts3)

    p1 = agg(g1, src2, dst2)

    g2 = pl.pallas_call(
        _dense2_body,
        grid=(nb,),
        in_specs=[
            pl.BlockSpec((NC, R, d), lambda i: (0, i, 0)),
            pl.BlockSpec((R, d), lambda i: (i, 0)),
            pl.BlockSpec((NC, R, 1), lambda i: (0, i, 0)),
            pl.BlockSpec((d, d), full(0, 0)),
            pl.BlockSpec((1, d), full(0, 0)),
        ],
        out_specs=pl.BlockSpec((R, d), lambda i: (i, 0)),
        out_shape=jax.ShapeDtypeStruct((n_pad, d), jnp.float32),
    )(p1, g1, parts3, W2, b1.reshape(1, d))

    p2 = agg(g2, src2, dst2)

    out = pl.pallas_call(
        _dense3_body,
        grid=(nb,),
        in_specs=[
            pl.BlockSpec((NC, R, d), lambda i: (0, i, 0)),
            pl.BlockSpec((R, d), lambda i: (i, 0)),
            pl.BlockSpec((NC, R, 1), lambda i: (0, i, 0)),
            pl.BlockSpec((1, d), full(0, 0)),
            pl.BlockSpec((1, d), full(0, 0)),
            pl.BlockSpec((1, 1), full(0, 0)),
        ],
        out_specs=pl.BlockSpec((R, 1), lambda i: (i, 0)),
        out_shape=jax.ShapeDtypeStruct((n_pad, 1), jnp.float32),
    )(p2, g2, parts3, b2.reshape(1, d), Wo.reshape(1, d), bo.reshape(1, 1))

    return out[:n, 0]


# speedup vs baseline: 1.2348x; 1.2348x over previous
import functools

import jax
import jax.numpy as jnp
from jax import lax
from jax.experimental import pallas as pl
from jax.experimental.pallas import tpu as pltpu
from jax.experimental.pallas import tpu_sc as plsc

NC = 2
NS = 16
NW = NC * NS
CHUNK = 80
R = 512


def _sc_mesh():
    return plsc.VectorSubcoreMesh(core_axis_name="c", subcore_axis_name="s")


BI = 25


DEG_BI = 5


def _make_deg_kernel(e_iters, n_pad):
    BI = DEG_BI
    iters = e_iters // NW
    nb = iters // BI
    rows_pt = n_pad // NS

    @functools.partial(
        pl.kernel,
        mesh=_sc_mesh(),
        out_type=jax.ShapeDtypeStruct((NC, NS, 1, rows_pt), jnp.float32),
        scratch_types=[
            pltpu.VMEM((BI, CHUNK), jnp.int32),
            pltpu.VMEM((CHUNK,), jnp.float32),
            pltpu.VMEM((rows_pt,), jnp.float32),
            pltpu.VMEM_SHARED((n_pad,), jnp.float32),
            pltpu.SemaphoreType.DMA,
            pltpu.SemaphoreType.DMA,
        ],
    )
    def deg_kernel(dst_hbm, out_hbm, dst_v, ones_v, stage_v, acc_sh, sem_a, sem_b):
        c = lax.axis_index("c")
        s = lax.axis_index("s")
        wid = c * NS + s

        def fill_ones(k, _):
            ones_v[pl.ds(k * 16, 16)] = jnp.ones((16,), jnp.float32)
            return 0

        lax.fori_loop(0, CHUNK // 16, fill_ones, 0)

        def zero_stage(k, _):
            stage_v[pl.ds(k * 16, 16)] = jnp.zeros((16,), jnp.float32)
            return 0

        lax.fori_loop(0, rows_pt // 16, zero_stage, 0)
        pltpu.sync_copy(stage_v, acc_sh.at[pl.ds(s * rows_pt, rows_pt)])
        plsc.subcore_barrier()

        def block(b, _):
            pltpu.sync_copy(dst_hbm.at[wid, b], dst_v)
            pltpu.async_copy(ones_v, acc_sh.at[dst_v.at[0]], sem_a, add=True)

            def body(t, __):
                j = 2 * t
                d1 = pltpu.async_copy(ones_v, acc_sh.at[dst_v.at[j + 1]],
                                      sem_b, add=True)
                pltpu.make_async_copy(ones_v, acc_sh.at[dst_v.at[j]], sem_a).wait()
                pltpu.async_copy(ones_v, acc_sh.at[dst_v.at[j + 2]], sem_a, add=True)
                d1.wait()
                return 0

            lax.fori_loop(0, (BI - 1) // 2, body, 0)
            pltpu.make_async_copy(ones_v, acc_sh.at[dst_v.at[BI - 1]], sem_a).wait()
            return 0

        lax.fori_loop(0, nb, block, 0)
        plsc.subcore_barrier()

        pltpu.sync_copy(acc_sh.at[pl.ds(s * rows_pt, rows_pt)], stage_v)
        pltpu.sync_copy(stage_v, out_hbm.at[c, s, 0])

    return deg_kernel


def _make_agg_kernel(e_iters, n_pad, d):
    iters = e_iters // NW
    rows_pt = n_pad // NS
    out_chunks = rows_pt // CHUNK

    nb = iters // BI

    DEPTH = 4

    @functools.partial(
        pl.kernel,
        mesh=_sc_mesh(),
        out_type=jax.ShapeDtypeStruct((NC, n_pad, d), jnp.float32),
        scratch_types=[
            pltpu.VMEM((BI, CHUNK), jnp.int32),
            pltpu.VMEM((BI, CHUNK), jnp.int32),
            pltpu.VMEM((DEPTH * CHUNK, d), jnp.float32),
            pltpu.VMEM_SHARED((n_pad, d), jnp.float32),
            pltpu.SemaphoreType.DMA((DEPTH,)),
        ],
    )
    def agg_kernel(g_hbm, src_hbm, dst_hbm, out_hbm,
                   src_v, dst_v, buf, acc_sh, sem):
        c = lax.axis_index("c")
        s = lax.axis_index("s")
        wid = c * NS + s
        r0 = s * rows_pt

        def slot_buf(j):
            off = pl.multiple_of(lax.rem(j, DEPTH) * CHUNK, CHUNK)
            return buf.at[pl.ds(off, CHUNK)]

        def zrow(r, _):
            def zcol(k, __):
                buf[r, pl.ds(k * 16, 16)] = jnp.zeros((16,), jnp.float32)
                return 0
            lax.fori_loop(0, d // 16, zcol, 0)
            return 0

        lax.fori_loop(0, CHUNK, zrow, 0)
        for k in range(out_chunks):
            pltpu.sync_copy(buf.at[pl.ds(0, CHUNK)],
                            acc_sh.at[pl.ds(r0 + k * CHUNK, CHUNK)])
        plsc.subcore_barrier()

        def block(b, _):
            pltpu.sync_copy(src_hbm.at[wid, b], src_v)
            pltpu.sync_copy(dst_hbm.at[wid, b], dst_v)
            for j in range(DEPTH - 1):
                pltpu.async_copy(g_hbm.at[src_v.at[j]], slot_buf(j),
                                 sem.at[j])

            def body(j, __):
                bj = slot_buf(j)
                pltpu.make_async_copy(g_hbm.at[src_v.at[j]], bj,
                                      sem.at[lax.rem(j, DEPTH)]).wait()

                @pl.when(j + DEPTH - 1 < BI)
                def _():
                    jn = j + DEPTH - 1
                    pltpu.async_copy(g_hbm.at[src_v.at[jn]], slot_buf(jn),
                                     sem.at[lax.rem(jn, DEPTH)])

                pltpu.sync_copy(bj, acc_sh.at[dst_v.at[j]], add=True)
                return 0

            lax.fori_loop(0, BI, body, 0)
            return 0

        lax.fori_loop(0, nb, block, 0)
        plsc.subcore_barrier()

        for k in range(out_chunks):
            pltpu.sync_copy(acc_sh.at[pl.ds(r0 + k * CHUNK, CHUNK)],
                            buf.at[pl.ds(0, CHUNK)])
            pltpu.sync_copy(buf.at[pl.ds(0, CHUNK)],
                            out_hbm.at[c, pl.ds(r0 + k * CHUNK, CHUNK)])

    return agg_kernel


def _dis_from_parts(p_ref):
    deg = p_ref[0, :, :] + p_ref[1, :, :] + 1.0
    return lax.rsqrt(deg)


def _dense1_body(x_ref, w_ref, p_ref, o_ref):
    dis = _dis_from_parts(p_ref)
    h = jnp.dot(x_ref[...], w_ref[...], preferred_element_type=jnp.float32)
    o_ref[...] = h * dis


def _dense2_body(pp_ref, g_ref, p_ref, w_ref, b_ref, o_ref):
    dis = _dis_from_parts(p_ref)
    z = (pp_ref[0, :, :] + pp_ref[1, :, :] + g_ref[...]) * dis + b_ref[...]
    a = jnp.maximum(z, 0.0)
    o_ref[...] = jnp.dot(a, w_ref[...], preferred_element_type=jnp.float32) * dis


def _dense3_body(pp_ref, g_ref, p_ref, b_ref, wo_ref, bo_ref, o_ref):
    dis = _dis_from_parts(p_ref)
    z = (pp_ref[0, :, :] + pp_ref[1, :, :] + g_ref[...]) * dis + b_ref[...]
    a = jnp.maximum(z, 0.0)
    t = jnp.sum(a * wo_ref[...], axis=1, keepdims=True) + bo_ref[...]
    o_ref[...] = jax.nn.sigmoid(t)


def kernel(x, edge_index, W1, b1, W2, b2, Wo, bo):
    n, d = x.shape
    e = edge_index.shape[1]
    n_pad = -(-n // R) * R
    assert n_pad % (NS * CHUNK) == 0 and BI % 2 == 1
    unit = NW * CHUNK * BI
    e_pad = -(-e // unit) * unit
    iters = e_pad // (NW * CHUNK)
    nb = n_pad // R

    src1, dst1 = edge_index[0], edge_index[1]
    if e_pad > e:
        assert n_pad > n
        pad_idx = (n + jnp.arange(e_pad - e, dtype=jnp.int32) % (n_pad - n))
        src1 = jnp.concatenate([src1, pad_idx])
        dst1 = jnp.concatenate([dst1, pad_idx])
    assert iters % BI == 0 and iters % DEG_BI == 0
    src2 = src1.reshape(NW, iters // BI, BI, CHUNK)
    dst2 = dst1.reshape(NW, iters // BI, BI, CHUNK)
    dst3 = dst1.reshape(NW, iters // DEG_BI, DEG_BI, CHUNK)
    x_pad = jnp.pad(x, ((0, n_pad - n), (0, 0)))

    parts = _make_deg_kernel(e_pad // CHUNK, n_pad)(dst3)
    parts3 = parts.reshape(NC, n_pad, 1)
    agg = _make_agg_kernel(e_pad // CHUNK, n_pad, d)

    full = lambda *idx: (lambda i: idx)
    g1 = pl.pallas_call(
        _dense1_body,
        grid=(nb,),
        in_specs=[
            pl.BlockSpec((R, d), lambda i: (i, 0)),
            pl.BlockSpec((d, d), full(0, 0)),
            pl.BlockSpec((NC, R, 1), lambda i: (0, i, 0)),
        ],
        out_specs=pl.BlockSpec((R, d), lambda i: (i, 0)),
        out_shape=jax.ShapeDtypeStruct((n_pad, d), jnp.float32),
    )(x_pad, W1, parts3)

    p1 = agg(g1, src2, dst2)

    g2 = pl.pallas_call(
        _dense2_body,
        grid=(nb,),
        in_specs=[
            pl.BlockSpec((NC, R, d), lambda i: (0, i, 0)),
            pl.BlockSpec((R, d), lambda i: (i, 0)),
            pl.BlockSpec((NC, R, 1), lambda i: (0, i, 0)),
            pl.BlockSpec((d, d), full(0, 0)),
            pl.BlockSpec((1, d), full(0, 0)),
        ],
        out_specs=pl.BlockSpec((R, d), lambda i: (i, 0)),
        out_shape=jax.ShapeDtypeStruct((n_pad, d), jnp.float32),
    )(p1, g1, parts3, W2, b1.reshape(1, d))

    p2 = agg(g2, src2, dst2)

    out = pl.pallas_call(
        _dense3_body,
        grid=(nb,),
        in_specs=[
            pl.BlockSpec((NC, R, d), lambda i: (0, i, 0)),
            pl.BlockSpec((R, d), lambda i: (i, 0)),
            pl.BlockSpec((NC, R, 1), lambda i: (0, i, 0)),
            pl.BlockSpec((1, d), full(0, 0)),
            pl.BlockSpec((1, d), full(0, 0)),
            pl.BlockSpec((1, 1), full(0, 0)),
        ],
        out_specs=pl.BlockSpec((R, 1), lambda i: (i, 0)),
        out_shape=jax.ShapeDtypeStruct((n_pad, 1), jnp.float32),
    )(p2, g2, parts3, b2.reshape(1, d), Wo.reshape(1, d), bo.reshape(1, 1))

    return out[:n, 0]
